# Initial kernel scaffold; baseline (speedup 1.0000x reference)
import jax
import jax.numpy as jnp
from jax.experimental import pallas as pl


def kernel(z, E):
    raise NotImplementedError("write your pallas kernel here")


# speedup vs baseline: 1.5308x; 1.5308x over previous
import functools

import jax
import jax.numpy as jnp
from jax import lax
from jax.experimental import pallas as pl
from jax.experimental.pallas import tpu as pltpu
from jax.experimental.pallas import tpu_sc as plsc

_BETA = 0.25
_K = 8192
_C = 32
_N = 16384
_KC = 1024
_NW = 32
_BPW = _N // _NW


def _argmin_body(z_ref, e_ref, idx_ref):
    zb = z_ref[0]
    sq = jnp.sum(zb * zb, axis=0, keepdims=True)
    run_min = jnp.full((1, 1024), jnp.inf, jnp.float32)
    run_idx = jnp.zeros((1, 1024), jnp.int32)
    iota = lax.broadcasted_iota(jnp.int32, (_KC, 1024), 0)
    zbb = zb.astype(jnp.bfloat16)
    for j in range(_K // _KC):
        ej = e_ref[j * _KC:(j + 1) * _KC, :]
        eqj = jnp.sum(ej * ej, axis=1, keepdims=True)
        dims = (((1,), (0,)), ((), ()))
        dot = lax.dot_general(ej.astype(jnp.bfloat16), zbb, dims,
                              preferred_element_type=jnp.float32)
        d2 = (sq + eqj) - 2.0 * dot
        d2m = jnp.maximum(d2, 0.0)
        d = jnp.where(d2m == 0.0, 0.0, d2m * lax.rsqrt(d2m))
        cmin = jnp.min(d, axis=0, keepdims=True)
        cidx = j * _KC + jnp.min(jnp.where(d == cmin, iota, _K),
                                 axis=0, keepdims=True)
        upd = cmin < run_min
        run_min = jnp.where(upd, cmin, run_min)
        run_idx = jnp.where(upd, cidx, run_idx)
    idx_ref[0] = run_idx


@functools.cache
def _make_sc_gather_out():
    mesh = plsc.VectorSubcoreMesh(core_axis_name="c", subcore_axis_name="s")

    @functools.partial(
        pl.kernel,
        mesh=mesh,
        out_type=[
            jax.ShapeDtypeStruct((_N * _C // 128, 128), jnp.float32),
            jax.ShapeDtypeStruct((_NW, 16), jnp.float32),
        ],
        scratch_types=[
            pltpu.VMEM((_BPW // 128, 128), jnp.int32),
            pltpu.VMEM((2, 128, 128), jnp.float32),
            pltpu.VMEM((_BPW * _C // 128, 128), jnp.float32),
            pltpu.VMEM((_BPW * _C // 128, 128), jnp.float32),
            pltpu.VMEM((16,), jnp.float32),
            pltpu.SemaphoreType.DMA,
            pltpu.SemaphoreType.DMA,
        ],
    )
    def _sc_gather_out(e_hbm, idx_hbm, z_hbm, out_hbm, loss_hbm,
                       idx_v, stage_v, z_v, out_v, part_v, sem0, sem1):
        wid = lax.axis_index("s") * 2 + lax.axis_index("c")
        nchunk = _BPW // 128
        zrows = _BPW * _C // 128
        sems = (sem0, sem1)
        pltpu.sync_copy(idx_hbm.at[pl.ds(wid * nchunk, nchunk)], idx_v)
        pltpu.sync_copy(z_hbm.at[pl.ds(wid * zrows, zrows)], z_v)

        def fire(j):
            return pltpu.async_copy(e_hbm.at[idx_v.at[j]],
                                    stage_v.at[j % 2], sems[j % 2])

        qrows = zrows // nchunk
        acc = jnp.zeros((16,), jnp.float32)
        cp = fire(0)
        for j in range(nchunk):
            nxt = fire(j + 1) if j + 1 < nchunk else None
            cp.wait()
            sbuf = stage_v.at[j % 2]

            def body(q, a, j=j, sbuf=sbuf):
                for t in range(8):
                    tok = q * 4 + t // 2
                    e = sbuf[tok, pl.ds((t % 2) * 16, 16)]
                    zv = z_v[j * qrows + q, pl.ds(t * 16, 16)]
                    dd = e - zv
                    out_v[j * qrows + q, pl.ds(t * 16, 16)] = zv - dd
                    a = a + dd * dd
                return a

            acc = lax.fori_loop(0, qrows, body, acc)
            cp = nxt

        pltpu.sync_copy(out_v, out_hbm.at[pl.ds(wid * zrows, zrows)])
        part_v[...] = acc
        pltpu.sync_copy(part_v, loss_hbm.at[wid])

    return _sc_gather_out


def kernel(z, E):
    B, C, H, W = z.shape
    z3 = z.reshape(B, C, H * W)
    idx3 = pl.pallas_call(
        _argmin_body,
        grid=(B,),
        in_specs=[
            pl.BlockSpec((1, C, H * W), lambda b: (b, 0, 0)),
            pl.BlockSpec((_K, _C), lambda b: (0, 0)),
        ],
        out_specs=pl.BlockSpec((1, 1, H * W), lambda b: (b, 0, 0)),
        out_shape=jax.ShapeDtypeStruct((B, 1, H * W), jnp.int32),
    )(z3, E)
    min_d = idx3.reshape(B, H * W)
    idx_rows = idx3.reshape(_N // 128, 128)
    z_flat = z.reshape(_N * _C // 128, 128)
    e_pad = jnp.pad(E, ((0, 0), (0, 128 - _C)))
    out_flat, parts = _make_sc_gather_out()(e_pad, idx_rows, z_flat)
    z_q_out = out_flat.reshape(B, C, H, W)
    m = jnp.sum(parts) / jnp.float32(_N * _C)
    loss = m + _BETA * m
    return (z_q_out, min_d, loss)
